# baseline (speedup 1.0000x reference)
import jax
import jax.numpy as jnp
from jax import lax
from jax.experimental import pallas as pl

PROMPT_LENGTH = 64
EMBED_SIZE = 2048
TILE = PROMPT_LENGTH


def _body(tokens_ref, prompt_ref, x_ref, out_ref):
    j = pl.program_id(1)

    @pl.when(j == 0)
    def _prefix():
        tok = tokens_ref[...]
        cols = lax.broadcasted_iota(jnp.int32, (PROMPT_LENGTH, PROMPT_LENGTH), 1)
        one_hot = (tok == cols).astype(jnp.float32)
        out_ref[0, :, :] = jnp.dot(one_hot, prompt_ref[...],
                                   preferred_element_type=jnp.float32)

    @pl.when(j != 0)
    def _bulk():
        out_ref[...] = x_ref[...]


def kernel(embedded_input, prompt_embedding, prompt_tokens):
    batch, seq_len, _ = embedded_input.shape
    tokens_2d = prompt_tokens.reshape(PROMPT_LENGTH, 1)
    n_tiles = (PROMPT_LENGTH + seq_len) // TILE

    return pl.pallas_call(
        _body,
        grid=(batch, n_tiles),
        in_specs=[
            pl.BlockSpec((PROMPT_LENGTH, 1), lambda b, j: (0, 0)),
            pl.BlockSpec((PROMPT_LENGTH, EMBED_SIZE), lambda b, j: (0, 0)),
            pl.BlockSpec((1, TILE, EMBED_SIZE),
                         lambda b, j: (b, jnp.maximum(j - 1, 0), 0)),
        ],
        out_specs=pl.BlockSpec((1, TILE, EMBED_SIZE), lambda b, j: (b, j, 0)),
        out_shape=jax.ShapeDtypeStruct(
            (batch, PROMPT_LENGTH + seq_len, EMBED_SIZE), jnp.float32),
    )(tokens_2d, prompt_embedding, embedded_input)
